# Initial kernel scaffold; baseline (speedup 1.0000x reference)
import jax
import jax.numpy as jnp
from jax.experimental import pallas as pl


def kernel(ns_emb, adj, adj_prior, condition, labels, Wh_post, Wq_post, Wk_post, Wv_post, Wo_post, Wmu_post, Wvar_post, Wh_prior, Wq_prior, Wk_prior, Wv_prior, Wo_prior, Wmu_prior, Wvar_prior, Wmap):
    raise NotImplementedError("write your pallas kernel here")


# speedup vs baseline: 1.0992x; 1.0992x over previous
import functools

import jax
import jax.numpy as jnp
from jax import lax
from jax.experimental import pallas as pl
from jax.experimental.pallas import tpu as pltpu

N = 1024
IN_DIM = 256
H1 = 128
H2 = 32
COND_LEN = 64
D_K = 64
MAX_K = 512


def _leaky(x):
    return jnp.where(x >= 0, x, 0.01 * x)


def _dot(a, b):
    return jax.lax.dot_general(a, b, (((1,), (0,)), ((), ())),
                               preferred_element_type=jnp.float32)


def _dot_t(a, b):
    return jax.lax.dot_general(a, b, (((1,), (1,)), ((), ())),
                               preferred_element_type=jnp.float32)


def _log_sigmoid(x):
    return jnp.minimum(x, 0.0) - jnp.log1p(jnp.exp(-jnp.abs(x)))


def _encode_block(ns_emb, adjm, cond, Wh, Wq, Wk, Wv, Wo, Wmu, Wvar):
    s = _leaky(_dot(ns_emb, Wh))
    hidden = _leaky(_dot(adjm, s))
    q = _dot(hidden, Wq)
    k = _dot(cond, Wk)
    v = _dot(cond, Wv)
    outs = []
    for h in range(2):
        sl = slice(h * D_K, (h + 1) * D_K)
        logits = _dot_t(q[:, sl], k[:, sl]) * 0.125
        mx = jnp.max(logits, axis=1, keepdims=True)
        e = jnp.exp(logits - mx)
        attn = e / jnp.sum(e, axis=1, keepdims=True)
        outs.append(_dot(attn, v[:, sl]))
    o = _dot(jnp.concatenate(outs, axis=1), Wo)
    mu = _leaky(_dot(adjm, _leaky(_dot(o, Wmu))))
    lv = _leaky(_dot(adjm, _leaky(_dot(o, Wvar))))
    return mu, lv


def _dense_body(ns_emb_ref, adj_ref, adjp_ref, cond_ref, labels_ref, eps_ref,
                Whp_ref, Wqp_ref, Wkp_ref, Wvp_ref, Wop_ref, Wmup_ref, Wvarp_ref,
                Whr_ref, Wqr_ref, Wkr_ref, Wvr_ref, Wor_ref, Wmur_ref, Wvarr_ref,
                Wmap_ref,
                scores_ref, m_ref, recons_ref, kld_ref):
    ns_emb = ns_emb_ref[...]
    cond = cond_ref[...]
    mu, logvar = _encode_block(ns_emb, adj_ref[...], cond,
                               Whp_ref[...], Wqp_ref[...], Wkp_ref[...],
                               Wvp_ref[...], Wop_ref[...], Wmup_ref[...],
                               Wvarp_ref[...])
    mu_p, logvar_p = _encode_block(ns_emb, adjp_ref[...], cond,
                                   Whr_ref[...], Wqr_ref[...], Wkr_ref[...],
                                   Wvr_ref[...], Wor_ref[...], Wmur_ref[...],
                                   Wvarr_ref[...])
    z = eps_ref[...] * jnp.exp(0.5 * logvar) + mu
    S = _dot_t(z, z)
    recover_adj = jax.nn.sigmoid(S)

    row = lax.broadcasted_iota(jnp.int32, (N, N), 0)
    col = lax.broadcasted_iota(jnp.int32, (N, N), 1)
    scores_ref[...] = jnp.where(col > row, recover_adj, 0.0)

    m_ref[...] = _leaky(_dot(ns_emb, Wmap_ref[...]))

    labels = labels_ref[...]
    s_sum = jnp.sum(labels)
    nf = jnp.float32(N)
    pos_weight = (nf * nf - s_sum + nf) / (s_sum - nf + 0.01)
    norm = nf * nf / (nf * nf - s_sum + nf)
    bce = -(pos_weight * labels * _log_sigmoid(recover_adj)
            + (1.0 - labels) * _log_sigmoid(-recover_adj))
    recons_ref[...] = jnp.reshape(norm * jnp.mean(bce), (1, 1))

    kld = 0.5 / nf * jnp.mean(jnp.sum(
        (mu_p - mu) ** 2 / jnp.exp(logvar_p)
        + jnp.exp(logvar) / jnp.exp(logvar_p)
        - 1.0 - (logvar - logvar_p), axis=1))
    kld_ref[...] = jnp.reshape(kld, (1, 1))


def _dense_call(ns_emb, adj, adj_prior, cond, labels, eps, *weights):
    return pl.pallas_call(
        _dense_body,
        out_shape=(
            jax.ShapeDtypeStruct((N, N), jnp.float32),
            jax.ShapeDtypeStruct((N, H2), jnp.float32),
            jax.ShapeDtypeStruct((1, 1), jnp.float32),
            jax.ShapeDtypeStruct((1, 1), jnp.float32),
        ),
    )(ns_emb, adj, adj_prior, cond, labels, eps, *weights)


def _assemble_body(vals_ref, idx_ref, m_ref, rel_ref):
    idx = idx_ref[...]
    r = idx // N
    c = idx % N
    cols = lax.broadcasted_iota(jnp.int32, (MAX_K, N), 1)
    oh_r = (r[:, None] == cols).astype(jnp.float32)
    oh_c = (c[:, None] == cols).astype(jnp.float32)
    m = m_ref[...]
    rel_ref[...] = _dot(oh_r, m) + _dot(oh_c, m)


def _assemble_call(vals, idx, m):
    return pl.pallas_call(
        _assemble_body,
        out_shape=jax.ShapeDtypeStruct((MAX_K, H2), jnp.float32),
    )(vals, idx, m)


def kernel(ns_emb, adj, adj_prior, condition, labels, Wh_post, Wq_post,
           Wk_post, Wv_post, Wo_post, Wmu_post, Wvar_post, Wh_prior, Wq_prior,
           Wk_prior, Wv_prior, Wo_prior, Wmu_prior, Wvar_prior, Wmap):
    cond = condition[0]
    eps = jax.random.normal(jax.random.key(42), (N, H2), dtype=jnp.float32)
    scores, m, recons, kld = _dense_call(
        ns_emb, adj, adj_prior, cond, labels, eps,
        Wh_post, Wq_post, Wk_post, Wv_post, Wo_post, Wmu_post, Wvar_post,
        Wh_prior, Wq_prior, Wk_prior, Wv_prior, Wo_prior, Wmu_prior,
        Wvar_prior, Wmap)

    vals, idx = jax.lax.top_k(scores.reshape(-1), MAX_K)

    relations = _assemble_call(vals, idx, m)
    rel_mask = jnp.zeros((MAX_K,), dtype=jnp.bool_)
    return relations, rel_mask, recons[0, 0], kld[0, 0]


# speedup vs baseline: 5.1079x; 4.6468x over previous
import functools

import jax
import jax.numpy as jnp
from jax import lax
from jax.experimental import pallas as pl
from jax.experimental.pallas import tpu as pltpu
from jax.experimental.pallas import tpu_sc as plsc

N = 1024
IN_DIM = 256
H1 = 128
H2 = 32
COND_LEN = 64
D_K = 64
MAX_K = 512


def _leaky(x):
    return jnp.where(x >= 0, x, 0.01 * x)


def _dot(a, b):
    return jax.lax.dot_general(a, b, (((1,), (0,)), ((), ())),
                               preferred_element_type=jnp.float32)


def _dot_t(a, b):
    return jax.lax.dot_general(a, b, (((1,), (1,)), ((), ())),
                               preferred_element_type=jnp.float32)


def _log_sigmoid(x):
    return jnp.minimum(x, 0.0) - jnp.log1p(jnp.exp(-jnp.abs(x)))


def _encode_block(ns_emb, adjm, cond, Wh, Wq, Wk, Wv, Wo, Wmu, Wvar):
    s = _leaky(_dot(ns_emb, Wh))
    hidden = _leaky(_dot(adjm, s))
    q = _dot(hidden, Wq)
    k = _dot(cond, Wk)
    v = _dot(cond, Wv)
    outs = []
    for h in range(2):
        sl = slice(h * D_K, (h + 1) * D_K)
        logits = _dot_t(q[:, sl], k[:, sl]) * 0.125
        mx = jnp.max(logits, axis=1, keepdims=True)
        e = jnp.exp(logits - mx)
        attn = e / jnp.sum(e, axis=1, keepdims=True)
        outs.append(_dot(attn, v[:, sl]))
    o = _dot(jnp.concatenate(outs, axis=1), Wo)
    mu = _leaky(_dot(adjm, _leaky(_dot(o, Wmu))))
    lv = _leaky(_dot(adjm, _leaky(_dot(o, Wvar))))
    return mu, lv


def _dense_body(ns_emb_ref, adj_ref, adjp_ref, cond_ref, labels_ref, eps_ref,
                Whp_ref, Wqp_ref, Wkp_ref, Wvp_ref, Wop_ref, Wmup_ref, Wvarp_ref,
                Whr_ref, Wqr_ref, Wkr_ref, Wvr_ref, Wor_ref, Wmur_ref, Wvarr_ref,
                Wmap_ref,
                scores_ref, m_ref, recons_ref, kld_ref):
    ns_emb = ns_emb_ref[...]
    cond = cond_ref[...]
    mu, logvar = _encode_block(ns_emb, adj_ref[...], cond,
                               Whp_ref[...], Wqp_ref[...], Wkp_ref[...],
                               Wvp_ref[...], Wop_ref[...], Wmup_ref[...],
                               Wvarp_ref[...])
    mu_p, logvar_p = _encode_block(ns_emb, adjp_ref[...], cond,
                                   Whr_ref[...], Wqr_ref[...], Wkr_ref[...],
                                   Wvr_ref[...], Wor_ref[...], Wmur_ref[...],
                                   Wvarr_ref[...])
    z = eps_ref[...] * jnp.exp(0.5 * logvar) + mu
    S = _dot_t(z, z)
    recover_adj = jax.nn.sigmoid(S)

    row = lax.broadcasted_iota(jnp.int32, (N, N), 0)
    col = lax.broadcasted_iota(jnp.int32, (N, N), 1)
    scores_ref[...] = lax.bitcast_convert_type(
        jnp.where(col > row, recover_adj, 0.0), jnp.int32)

    m_ref[...] = _leaky(_dot(ns_emb, Wmap_ref[...]))

    labels = labels_ref[...]
    s_sum = jnp.sum(labels)
    nf = jnp.float32(N)
    pos_weight = (nf * nf - s_sum + nf) / (s_sum - nf + 0.01)
    norm = nf * nf / (nf * nf - s_sum + nf)
    bce = -(pos_weight * labels * _log_sigmoid(recover_adj)
            + (1.0 - labels) * _log_sigmoid(-recover_adj))
    recons_ref[...] = jnp.reshape(norm * jnp.mean(bce), (1, 1))

    kld = 0.5 / nf * jnp.mean(jnp.sum(
        (mu_p - mu) ** 2 / jnp.exp(logvar_p)
        + jnp.exp(logvar) / jnp.exp(logvar_p)
        - 1.0 - (logvar - logvar_p), axis=1))
    kld_ref[...] = jnp.reshape(kld, (1, 1))


def _dense_call(ns_emb, adj, adj_prior, cond, labels, eps, *weights):
    return pl.pallas_call(
        _dense_body,
        out_shape=(
            jax.ShapeDtypeStruct((N, N), jnp.int32),
            jax.ShapeDtypeStruct((N, H2), jnp.float32),
            jax.ShapeDtypeStruct((1, 1), jnp.float32),
            jax.ShapeDtypeStruct((1, 1), jnp.float32),
        ),
    )(ns_emb, adj, adj_prior, cond, labels, eps, *weights)


NC = 2
NS = 16
L = 16
TOT = N * N
HALF = TOT // NC
CHUNK = HALF // NS
NVR = CHUNK // L
HBINS = 256
BUF = 544


def _sc_topk_body(scores_hbm, outv_hbm, outi_hbm,
                  chunk_v, hist_v, red_v, gall_v, ghist_v,
                  gtv_v, gti_v, tie_v, cnt_v, cntall_v,
                  tmpv_v, tmpi_v, cbufv_v, cbufi_v,
                  shist_s, sgtv_s, sgti_s, stie_s, scnt_s):
    c = lax.axis_index("c")
    s = lax.axis_index("s")
    base = c * HALF + s * CHUNK
    lane = lax.iota(jnp.int32, L)
    ones_i = jnp.ones((L,), jnp.int32)

    pltpu.sync_copy(scores_hbm.at[pl.ds(base, CHUNK)], chunk_v)

    prefix = jnp.int32(0)
    kp = jnp.int32(MAX_K)

    for rnd in range(4):
        shift = 24 - 8 * rnd

        def zero_body(i, _):
            hist_v[pl.ds(i * L, L)] = jnp.zeros((L,), jnp.int32)
            return 0
        lax.fori_loop(0, (NS * HBINS) // L, zero_body, 0)

        def scan_body(j, _):
            bits = chunk_v[pl.ds(j * L, L)]
            digit = (bits >> shift) & (HBINS - 1)
            addr = lane * HBINS + digit
            if rnd == 0:
                plsc.addupdate_scatter(hist_v, [addr], ones_i)
            else:
                match = (bits >> (shift + 8)) == prefix
                plsc.addupdate_scatter(hist_v, [addr], ones_i, mask=match)
            return 0
        lax.fori_loop(0, NVR, scan_body, 0)

        def red_body(g, _):
            acc = hist_v[pl.ds(g * L, L)]
            for l in range(1, NS):
                acc = acc + hist_v[pl.ds(l * HBINS + g * L, L)]
            red_v[pl.ds(g * L, L)] = acc
            return 0
        lax.fori_loop(0, HBINS // L, red_body, 0)

        pltpu.sync_copy(red_v, shist_s.at[s])
        plsc.subcore_barrier()
        pltpu.sync_copy(shist_s, gall_v)
        plsc.subcore_barrier()

        def sum_body(g, _):
            acc = gall_v[0, pl.ds(g * L, L)]
            for l in range(1, NS):
                acc = acc + gall_v[l, pl.ds(g * L, L)]
            ghist_v[pl.ds(g * L, L)] = acc
            return 0
        lax.fori_loop(0, HBINS // L, sum_body, 0)

        def sel_body(b, carry):
            cum, bstar, cumsel, found = carry
            d = (HBINS - 1) - b
            cnt = ghist_v[pl.ds(d, L)][0]
            newcum = cum + cnt
            take = jnp.logical_and(found == 0, newcum >= kp)
            bstar = jnp.where(take, d, bstar)
            cumsel = jnp.where(take, cum, cumsel)
            found = jnp.where(take, jnp.int32(1), found)
            return newcum, bstar, cumsel, found

        _, bstar, cumsel, _ = lax.fori_loop(
            0, HBINS, sel_body,
            (jnp.int32(0), jnp.int32(0), jnp.int32(0), jnp.int32(0)))
        kp = kp - cumsel
        prefix = (prefix << 8) | bstar

    vstar = prefix
    vstar_vec = jnp.full((L,), 1, jnp.int32) * vstar

    def coll_body(j, carry):
        off_gt, off_tie = carry
        bits = chunk_v[pl.ds(j * L, L)]
        gidx = base + j * L + lane
        gt = bits > vstar
        tie = bits == vstar

        @pl.when(off_gt < MAX_K)
        def _():
            plsc.store_compressed(gtv_v.at[pl.ds(off_gt, L)], bits, mask=gt)
            plsc.store_compressed(gti_v.at[pl.ds(off_gt, L)], gidx, mask=gt)

        @pl.when(off_tie < MAX_K)
        def _():
            plsc.store_compressed(tie_v.at[pl.ds(off_tie, L)], gidx, mask=tie)

        n_gt = jnp.sum(gt.astype(jnp.int32))
        n_tie = jnp.sum(tie.astype(jnp.int32))
        off_gt = jnp.where(off_gt < MAX_K, off_gt + n_gt, off_gt)
        off_tie = jnp.where(off_tie < MAX_K, off_tie + n_tie, off_tie)
        return off_gt, off_tie

    off_gt, off_tie = lax.fori_loop(0, NVR, coll_body,
                                    (jnp.int32(0), jnp.int32(0)))

    cnt_v[...] = jnp.where(lane == 0, off_gt,
                           jnp.where(lane == 1, off_tie, 0))
    pltpu.sync_copy(gtv_v, sgtv_s.at[s])
    pltpu.sync_copy(gti_v, sgti_s.at[s])
    pltpu.sync_copy(tie_v, stie_s.at[s])
    pltpu.sync_copy(cnt_v, scnt_s.at[s])
    plsc.subcore_barrier()

    @pl.when(s == 0)
    def _():
        pltpu.sync_copy(scnt_s, cntall_v)
        off = jnp.int32(0)
        for w in range(NS):
            pltpu.sync_copy(sgtv_s.at[w], tmpv_v)
            pltpu.sync_copy(sgti_s.at[w], tmpi_v)
            cw = cntall_v[w, pl.ds(0, L)][0]

            def gt_body(j, off):
                valid = lane < (cw - j * L)
                plsc.store_compressed(cbufv_v.at[pl.ds(off, L)],
                                      tmpv_v[pl.ds(j * L, L)], mask=valid)
                plsc.store_compressed(cbufi_v.at[pl.ds(off, L)],
                                      tmpi_v[pl.ds(j * L, L)], mask=valid)
                return off + jnp.sum(valid.astype(jnp.int32))
            off = lax.fori_loop(0, BUF // L, gt_body, off)

        taken = jnp.int32(0)
        for w in range(NS):
            pltpu.sync_copy(stie_s.at[w], tmpi_v)
            cw = cntall_v[w, pl.ds(0, L)][1]

            def tie_body(j, carry):
                off, taken = carry
                valid = jnp.logical_and(lane < (cw - j * L),
                                        (taken + lane) < kp)
                plsc.store_compressed(cbufv_v.at[pl.ds(off, L)],
                                      vstar_vec, mask=valid)
                plsc.store_compressed(cbufi_v.at[pl.ds(off, L)],
                                      tmpi_v[pl.ds(j * L, L)], mask=valid)
                n = jnp.sum(valid.astype(jnp.int32))
                return off + n, taken + n
            off, taken = lax.fori_loop(0, BUF // L, tie_body, (off, taken))

        pltpu.sync_copy(cbufv_v.at[pl.ds(0, MAX_K)], outv_hbm.at[c])
        pltpu.sync_copy(cbufi_v.at[pl.ds(0, MAX_K)], outi_hbm.at[c])


def _sc_topk(scores_flat):
    mesh = plsc.VectorSubcoreMesh(core_axis_name="c", subcore_axis_name="s",
                                  num_cores=NC, num_subcores=NS)
    f = pl.kernel(
        _sc_topk_body,
        out_type=(
            jax.ShapeDtypeStruct((NC, MAX_K), jnp.int32),
            jax.ShapeDtypeStruct((NC, MAX_K), jnp.int32),
        ),
        mesh=mesh,
        compiler_params=pltpu.CompilerParams(needs_layout_passes=False),
        scratch_types=[
            pltpu.VMEM((CHUNK,), jnp.int32),
            pltpu.VMEM((NS * HBINS,), jnp.int32),
            pltpu.VMEM((HBINS,), jnp.int32),
            pltpu.VMEM((NS, HBINS), jnp.int32),
            pltpu.VMEM((HBINS + L,), jnp.int32),
            pltpu.VMEM((BUF,), jnp.int32),
            pltpu.VMEM((BUF,), jnp.int32),
            pltpu.VMEM((BUF,), jnp.int32),
            pltpu.VMEM((L,), jnp.int32),
            pltpu.VMEM((NS, L), jnp.int32),
            pltpu.VMEM((BUF,), jnp.int32),
            pltpu.VMEM((BUF,), jnp.int32),
            pltpu.VMEM((BUF,), jnp.int32),
            pltpu.VMEM((BUF,), jnp.int32),
            pltpu.VMEM_SHARED((NS, HBINS), jnp.int32),
            pltpu.VMEM_SHARED((NS, BUF), jnp.int32),
            pltpu.VMEM_SHARED((NS, BUF), jnp.int32),
            pltpu.VMEM_SHARED((NS, BUF), jnp.int32),
            pltpu.VMEM_SHARED((NS, L), jnp.int32),
        ],
    )
    return f(scores_flat)


def _final_body(vals_ref, idx_ref, m_ref, rel_ref):
    vals = vals_ref[...]
    idxf = idx_ref[...].astype(jnp.float32)
    eye = (lax.broadcasted_iota(jnp.int32, (128, 128), 0)
           == lax.broadcasted_iota(jnp.int32, (128, 128), 1)).astype(jnp.float32)

    def _tcol(row):
        return lax.dot_general(eye, row, (((1,), (1,)), ((), ())),
                               preferred_element_type=jnp.float32)

    colv = [_tcol(vals[j:j + 1, :]) for j in range(8)]
    coli = [_tcol(idxf[j:j + 1, :]) for j in range(8)]

    p_f = lax.broadcasted_iota(jnp.int32, (MAX_K, 1), 0).astype(jnp.float32)
    fidx = jnp.zeros((MAX_K, 1), jnp.float32)
    for i in range(8):
        vi = vals[i:i + 1, :]
        ii = idxf[i:i + 1, :]
        acc = jnp.zeros((1, 128), jnp.float32)
        for j in range(8):
            ahead = jnp.logical_or(
                colv[j] > vi,
                jnp.logical_and(colv[j] == vi, coli[j] < ii))
            acc = acc + jnp.sum(ahead.astype(jnp.float32), axis=0,
                                keepdims=True)
        sel = (acc == p_f).astype(jnp.float32)
        fidx = fidx + jnp.sum(sel * ii, axis=1, keepdims=True)

    iidx = fidx.astype(jnp.int32)
    q = lax.broadcasted_iota(jnp.int32, (MAX_K, N), 1)
    oh_r = ((iidx >> 10) == q).astype(jnp.float32)
    oh_c = ((iidx & (N - 1)) == q).astype(jnp.float32)
    m = m_ref[...]
    rel_ref[...] = _dot(oh_r, m) + _dot(oh_c, m)


def _final_call(vals8, idx8, m):
    return pl.pallas_call(
        _final_body,
        out_shape=jax.ShapeDtypeStruct((MAX_K, H2), jnp.float32),
    )(vals8, idx8, m)


def kernel(ns_emb, adj, adj_prior, condition, labels, Wh_post, Wq_post,
           Wk_post, Wv_post, Wo_post, Wmu_post, Wvar_post, Wh_prior, Wq_prior,
           Wk_prior, Wv_prior, Wo_prior, Wmu_prior, Wvar_prior, Wmap):
    cond = condition[0]
    eps = jax.random.normal(jax.random.key(42), (N, H2), dtype=jnp.float32)
    scores, m, recons, kld = _dense_call(
        ns_emb, adj, adj_prior, cond, labels, eps,
        Wh_post, Wq_post, Wk_post, Wv_post, Wo_post, Wmu_post, Wvar_post,
        Wh_prior, Wq_prior, Wk_prior, Wv_prior, Wo_prior, Wmu_prior,
        Wvar_prior, Wmap)

    outv, outi = _sc_topk(scores.reshape(-1))
    vals8 = lax.bitcast_convert_type(outv, jnp.float32).reshape(8, 128)
    relations = _final_call(vals8, outi.reshape(8, 128), m)
    rel_mask = jnp.zeros((MAX_K,), dtype=jnp.bool_)
    return relations, rel_mask, recons[0, 0], kld[0, 0]


# speedup vs baseline: 6.2858x; 1.2306x over previous
import functools

import jax
import jax.numpy as jnp
from jax import lax
from jax.experimental import pallas as pl
from jax.experimental.pallas import tpu as pltpu
from jax.experimental.pallas import tpu_sc as plsc

N = 1024
IN_DIM = 256
H1 = 128
H2 = 32
COND_LEN = 64
D_K = 64
MAX_K = 512


def _leaky(x):
    return jnp.where(x >= 0, x, 0.01 * x)


def _dot(a, b):
    return jax.lax.dot_general(a, b, (((1,), (0,)), ((), ())),
                               preferred_element_type=jnp.float32)


def _dot_t(a, b):
    return jax.lax.dot_general(a, b, (((1,), (1,)), ((), ())),
                               preferred_element_type=jnp.float32)


def _log_sigmoid(x):
    return jnp.minimum(x, 0.0) - jnp.log1p(jnp.exp(-jnp.abs(x)))


def _encode_block(ns_emb, adjm, cond, Wh, Wq, Wk, Wv, Wo, Wmu, Wvar):
    s = _leaky(_dot(ns_emb, Wh))
    hidden = _leaky(_dot(adjm, s))
    q = _dot(hidden, Wq)
    k = _dot(cond, Wk)
    v = _dot(cond, Wv)
    outs = []
    for h in range(2):
        sl = slice(h * D_K, (h + 1) * D_K)
        logits = _dot_t(q[:, sl], k[:, sl]) * 0.125
        mx = jnp.max(logits, axis=1, keepdims=True)
        e = jnp.exp(logits - mx)
        attn = e / jnp.sum(e, axis=1, keepdims=True)
        outs.append(_dot(attn, v[:, sl]))
    o = _dot(jnp.concatenate(outs, axis=1), Wo)
    mu = _leaky(_dot(adjm, _leaky(_dot(o, Wmu))))
    lv = _leaky(_dot(adjm, _leaky(_dot(o, Wvar))))
    return mu, lv


def _dense_body(ns_emb_ref, adj_ref, adjp_ref, cond_ref, labels_ref, eps_ref,
                Whp_ref, Wqp_ref, Wkp_ref, Wvp_ref, Wop_ref, Wmup_ref, Wvarp_ref,
                Whr_ref, Wqr_ref, Wkr_ref, Wvr_ref, Wor_ref, Wmur_ref, Wvarr_ref,
                Wmap_ref,
                scores_ref, m_ref, recons_ref, kld_ref):
    ns_emb = ns_emb_ref[...]
    cond = cond_ref[...]
    mu, logvar = _encode_block(ns_emb, adj_ref[...], cond,
                               Whp_ref[...], Wqp_ref[...], Wkp_ref[...],
                               Wvp_ref[...], Wop_ref[...], Wmup_ref[...],
                               Wvarp_ref[...])
    mu_p, logvar_p = _encode_block(ns_emb, adjp_ref[...], cond,
                                   Whr_ref[...], Wqr_ref[...], Wkr_ref[...],
                                   Wvr_ref[...], Wor_ref[...], Wmur_ref[...],
                                   Wvarr_ref[...])
    z = eps_ref[...] * jnp.exp(0.5 * logvar) + mu
    S = _dot_t(z, z)
    recover_adj = jax.nn.sigmoid(S)

    row = lax.broadcasted_iota(jnp.int32, (N, N), 0)
    col = lax.broadcasted_iota(jnp.int32, (N, N), 1)
    scores_ref[...] = lax.bitcast_convert_type(
        jnp.where(col > row, recover_adj, 0.0), jnp.int32)

    m_ref[...] = _leaky(_dot(ns_emb, Wmap_ref[...]))

    labels = labels_ref[...]
    s_sum = jnp.sum(labels)
    nf = jnp.float32(N)
    pos_weight = (nf * nf - s_sum + nf) / (s_sum - nf + 0.01)
    norm = nf * nf / (nf * nf - s_sum + nf)
    bce = -(pos_weight * labels * _log_sigmoid(recover_adj)
            + (1.0 - labels) * _log_sigmoid(-recover_adj))
    recons_ref[...] = jnp.reshape(norm * jnp.mean(bce), (1, 1))

    kld = 0.5 / nf * jnp.mean(jnp.sum(
        (mu_p - mu) ** 2 / jnp.exp(logvar_p)
        + jnp.exp(logvar) / jnp.exp(logvar_p)
        - 1.0 - (logvar - logvar_p), axis=1))
    kld_ref[...] = jnp.reshape(kld, (1, 1))


def _dense_call(ns_emb, adj, adj_prior, cond, labels, eps, *weights):
    return pl.pallas_call(
        _dense_body,
        out_shape=(
            jax.ShapeDtypeStruct((N, N), jnp.int32),
            jax.ShapeDtypeStruct((N, H2), jnp.float32),
            jax.ShapeDtypeStruct((1, 1), jnp.float32),
            jax.ShapeDtypeStruct((1, 1), jnp.float32),
        ),
    )(ns_emb, adj, adj_prior, cond, labels, eps, *weights)


NC = 2
NS = 16
L = 16
TOT = N * N
HALF = TOT // NC
CHUNK = HALF // NS
NVR = CHUNK // L
HBINS = 256
BUF = 544


def _sc_topk_body(scores_hbm, outv_hbm, outi_hbm,
                  chunk_v, candb_v, candi_v, hist_v, red_v, gall_v, ghist_v,
                  gtv_v, gti_v, tie_v, cnt_v, cntall_v,
                  tmpv_v, tmpi_v, cbufv_v, cbufi_v,
                  shist_s, sgtv_s, sgti_s, stie_s, scnt_s):
    c = lax.axis_index("c")
    s = lax.axis_index("s")
    base = c * HALF + s * CHUNK
    lane = lax.iota(jnp.int32, L)
    ones_i = jnp.ones((L,), jnp.int32)

    pltpu.sync_copy(scores_hbm.at[pl.ds(base, CHUNK)], chunk_v)

    prefix = jnp.int32(0)
    kp = jnp.int32(MAX_K)
    mcnt = jnp.int32(0)

    for rnd in range(4):
        shift = 24 - 8 * rnd

        def zero_body(i, _):
            hist_v[pl.ds(i * L, L)] = jnp.zeros((L,), jnp.int32)
            return 0
        lax.fori_loop(0, (NS * HBINS) // L, zero_body, 0, unroll=8)

        if rnd == 0:
            def scan_body(j, _):
                bits = chunk_v[pl.ds(j * L, L)]
                digit = (bits >> shift) & (HBINS - 1)
                addr = lane * HBINS + digit
                plsc.addupdate_scatter(hist_v, [addr], ones_i)
                return 0
            lax.fori_loop(0, NVR, scan_body, 0, unroll=8)
        else:
            def scan_body(j, _):
                bits = candb_v[pl.ds(j * L, L)]
                digit = (bits >> shift) & (HBINS - 1)
                addr = lane * HBINS + digit
                match = jnp.logical_and((bits >> (shift + 8)) == prefix,
                                        (j * L + lane) < mcnt)
                plsc.addupdate_scatter(hist_v, [addr], ones_i, mask=match)
                return 0
            lax.fori_loop(0, (mcnt + L - 1) >> 4, scan_body, 0)

        def red_body(g, _):
            acc = hist_v[pl.ds(g * L, L)]
            for l in range(1, NS):
                acc = acc + hist_v[pl.ds(l * HBINS + g * L, L)]
            red_v[pl.ds(g * L, L)] = acc
            return 0
        lax.fori_loop(0, HBINS // L, red_body, 0)

        pltpu.sync_copy(red_v, shist_s.at[s])
        plsc.subcore_barrier()
        pltpu.sync_copy(shist_s, gall_v)
        plsc.subcore_barrier()

        def sum_body(g, _):
            acc = gall_v[0, pl.ds(g * L, L)]
            for l in range(1, NS):
                acc = acc + gall_v[l, pl.ds(g * L, L)]
            ghist_v[pl.ds(g * L, L)] = acc
            return 0
        lax.fori_loop(0, HBINS // L, sum_body, 0)

        def sel_body(b, carry):
            cum, bstar, cumsel, found = carry
            d = (HBINS - 1) - b
            cnt = ghist_v[pl.ds(d, L)][0]
            newcum = cum + cnt
            take = jnp.logical_and(found == 0, newcum >= kp)
            bstar = jnp.where(take, d, bstar)
            cumsel = jnp.where(take, cum, cumsel)
            found = jnp.where(take, jnp.int32(1), found)
            return newcum, bstar, cumsel, found

        _, bstar, cumsel, _ = lax.fori_loop(
            0, HBINS, sel_body,
            (jnp.int32(0), jnp.int32(0), jnp.int32(0), jnp.int32(0)))
        kp = kp - cumsel
        prefix = (prefix << 8) | bstar

        if rnd == 0:
            def comp_body(j, mc):
                bits = chunk_v[pl.ds(j * L, L)]
                keep = (bits >> 24) >= prefix
                gidx = base + j * L + lane
                plsc.store_compressed(candb_v.at[pl.ds(mc, L)], bits,
                                      mask=keep)
                plsc.store_compressed(candi_v.at[pl.ds(mc, L)], gidx,
                                      mask=keep)
                return mc + jnp.sum(keep.astype(jnp.int32))
            mcnt = lax.fori_loop(0, NVR, comp_body, jnp.int32(0), unroll=4)

    vstar = prefix
    vstar_vec = jnp.full((L,), 1, jnp.int32) * vstar

    def coll_body(j, carry):
        off_gt, off_tie = carry
        bits = candb_v[pl.ds(j * L, L)]
        gidx = candi_v[pl.ds(j * L, L)]
        valid = (j * L + lane) < mcnt
        gt = jnp.logical_and(valid, bits > vstar)
        tie = jnp.logical_and(valid, bits == vstar)

        @pl.when(off_gt < MAX_K)
        def _():
            plsc.store_compressed(gtv_v.at[pl.ds(off_gt, L)], bits, mask=gt)
            plsc.store_compressed(gti_v.at[pl.ds(off_gt, L)], gidx, mask=gt)

        @pl.when(off_tie < MAX_K)
        def _():
            plsc.store_compressed(tie_v.at[pl.ds(off_tie, L)], gidx, mask=tie)

        n_gt = jnp.sum(gt.astype(jnp.int32))
        n_tie = jnp.sum(tie.astype(jnp.int32))
        off_gt = jnp.where(off_gt < MAX_K, off_gt + n_gt, off_gt)
        off_tie = jnp.where(off_tie < MAX_K, off_tie + n_tie, off_tie)
        return off_gt, off_tie

    off_gt, off_tie = lax.fori_loop(0, (mcnt + L - 1) >> 4, coll_body,
                                    (jnp.int32(0), jnp.int32(0)))

    cnt_v[...] = jnp.where(lane == 0, off_gt,
                           jnp.where(lane == 1, off_tie, 0))
    pltpu.sync_copy(gtv_v, sgtv_s.at[s])
    pltpu.sync_copy(gti_v, sgti_s.at[s])
    pltpu.sync_copy(tie_v, stie_s.at[s])
    pltpu.sync_copy(cnt_v, scnt_s.at[s])
    plsc.subcore_barrier()

    @pl.when(s == 0)
    def _():
        pltpu.sync_copy(scnt_s, cntall_v)
        off = jnp.int32(0)
        for w in range(NS):
            pltpu.sync_copy(sgtv_s.at[w], tmpv_v)
            pltpu.sync_copy(sgti_s.at[w], tmpi_v)
            cw = cntall_v[w, pl.ds(0, L)][0]

            def gt_body(j, off):
                valid = lane < (cw - j * L)
                plsc.store_compressed(cbufv_v.at[pl.ds(off, L)],
                                      tmpv_v[pl.ds(j * L, L)], mask=valid)
                plsc.store_compressed(cbufi_v.at[pl.ds(off, L)],
                                      tmpi_v[pl.ds(j * L, L)], mask=valid)
                return off + jnp.sum(valid.astype(jnp.int32))
            off = lax.fori_loop(0, (cw + L - 1) >> 4, gt_body, off)

        taken = jnp.int32(0)
        for w in range(NS):
            pltpu.sync_copy(stie_s.at[w], tmpi_v)
            cw = cntall_v[w, pl.ds(0, L)][1]

            def tie_body(j, carry):
                off, taken = carry
                valid = jnp.logical_and(lane < (cw - j * L),
                                        (taken + lane) < kp)
                plsc.store_compressed(cbufv_v.at[pl.ds(off, L)],
                                      vstar_vec, mask=valid)
                plsc.store_compressed(cbufi_v.at[pl.ds(off, L)],
                                      tmpi_v[pl.ds(j * L, L)], mask=valid)
                n = jnp.sum(valid.astype(jnp.int32))
                return off + n, taken + n
            off, taken = lax.fori_loop(0, (cw + L - 1) >> 4, tie_body,
                                       (off, taken))

        pltpu.sync_copy(cbufv_v.at[pl.ds(0, MAX_K)], outv_hbm.at[c])
        pltpu.sync_copy(cbufi_v.at[pl.ds(0, MAX_K)], outi_hbm.at[c])


def _sc_topk(scores_flat):
    mesh = plsc.VectorSubcoreMesh(core_axis_name="c", subcore_axis_name="s",
                                  num_cores=NC, num_subcores=NS)
    f = pl.kernel(
        _sc_topk_body,
        out_type=(
            jax.ShapeDtypeStruct((NC, MAX_K), jnp.int32),
            jax.ShapeDtypeStruct((NC, MAX_K), jnp.int32),
        ),
        mesh=mesh,
        compiler_params=pltpu.CompilerParams(needs_layout_passes=False),
        scratch_types=[
            pltpu.VMEM((CHUNK,), jnp.int32),
            pltpu.VMEM((CHUNK + L,), jnp.int32),
            pltpu.VMEM((CHUNK + L,), jnp.int32),
            pltpu.VMEM((NS * HBINS,), jnp.int32),
            pltpu.VMEM((HBINS,), jnp.int32),
            pltpu.VMEM((NS, HBINS), jnp.int32),
            pltpu.VMEM((HBINS + L,), jnp.int32),
            pltpu.VMEM((BUF,), jnp.int32),
            pltpu.VMEM((BUF,), jnp.int32),
            pltpu.VMEM((BUF,), jnp.int32),
            pltpu.VMEM((L,), jnp.int32),
            pltpu.VMEM((NS, L), jnp.int32),
            pltpu.VMEM((BUF,), jnp.int32),
            pltpu.VMEM((BUF,), jnp.int32),
            pltpu.VMEM((BUF,), jnp.int32),
            pltpu.VMEM((BUF,), jnp.int32),
            pltpu.VMEM_SHARED((NS, HBINS), jnp.int32),
            pltpu.VMEM_SHARED((NS, BUF), jnp.int32),
            pltpu.VMEM_SHARED((NS, BUF), jnp.int32),
            pltpu.VMEM_SHARED((NS, BUF), jnp.int32),
            pltpu.VMEM_SHARED((NS, L), jnp.int32),
        ],
    )
    return f(scores_flat)


def _final_body(vals_ref, idx_ref, m_ref, rel_ref):
    vals = vals_ref[...]
    idxf = idx_ref[...].astype(jnp.float32)
    eye = (lax.broadcasted_iota(jnp.int32, (128, 128), 0)
           == lax.broadcasted_iota(jnp.int32, (128, 128), 1)).astype(jnp.float32)

    def _tcol(row):
        return lax.dot_general(eye, row, (((1,), (1,)), ((), ())),
                               preferred_element_type=jnp.float32)

    colv = [_tcol(vals[j:j + 1, :]) for j in range(8)]
    coli = [_tcol(idxf[j:j + 1, :]) for j in range(8)]

    p_f = lax.broadcasted_iota(jnp.int32, (MAX_K, 1), 0).astype(jnp.float32)
    fidx = jnp.zeros((MAX_K, 1), jnp.float32)
    for i in range(8):
        vi = vals[i:i + 1, :]
        ii = idxf[i:i + 1, :]
        acc = jnp.zeros((1, 128), jnp.float32)
        for j in range(8):
            ahead = jnp.logical_or(
                colv[j] > vi,
                jnp.logical_and(colv[j] == vi, coli[j] < ii))
            acc = acc + jnp.sum(ahead.astype(jnp.float32), axis=0,
                                keepdims=True)
        sel = (acc == p_f).astype(jnp.float32)
        fidx = fidx + jnp.sum(sel * ii, axis=1, keepdims=True)

    iidx = fidx.astype(jnp.int32)
    q = lax.broadcasted_iota(jnp.int32, (MAX_K, N), 1)
    oh_r = ((iidx >> 10) == q).astype(jnp.float32)
    oh_c = ((iidx & (N - 1)) == q).astype(jnp.float32)
    m = m_ref[...]
    rel_ref[...] = _dot(oh_r, m) + _dot(oh_c, m)


def _final_call(vals8, idx8, m):
    return pl.pallas_call(
        _final_body,
        out_shape=jax.ShapeDtypeStruct((MAX_K, H2), jnp.float32),
    )(vals8, idx8, m)


def kernel(ns_emb, adj, adj_prior, condition, labels, Wh_post, Wq_post,
           Wk_post, Wv_post, Wo_post, Wmu_post, Wvar_post, Wh_prior, Wq_prior,
           Wk_prior, Wv_prior, Wo_prior, Wmu_prior, Wvar_prior, Wmap):
    cond = condition[0]
    eps = jax.random.normal(jax.random.key(42), (N, H2), dtype=jnp.float32)
    scores, m, recons, kld = _dense_call(
        ns_emb, adj, adj_prior, cond, labels, eps,
        Wh_post, Wq_post, Wk_post, Wv_post, Wo_post, Wmu_post, Wvar_post,
        Wh_prior, Wq_prior, Wk_prior, Wv_prior, Wo_prior, Wmu_prior,
        Wvar_prior, Wmap)

    outv, outi = _sc_topk(scores.reshape(-1))
    vals8 = lax.bitcast_convert_type(outv, jnp.float32).reshape(8, 128)
    relations = _final_call(vals8, outi.reshape(8, 128), m)
    rel_mask = jnp.zeros((MAX_K,), dtype=jnp.bool_)
    return relations, rel_mask, recons[0, 0], kld[0, 0]


# speedup vs baseline: 6.6921x; 1.0646x over previous
import functools

import jax
import jax.numpy as jnp
from jax import lax
from jax.experimental import pallas as pl
from jax.experimental.pallas import tpu as pltpu
from jax.experimental.pallas import tpu_sc as plsc

N = 1024
IN_DIM = 256
H1 = 128
H2 = 32
COND_LEN = 64
D_K = 64
MAX_K = 512


def _leaky(x):
    return jnp.where(x >= 0, x, 0.01 * x)


def _dot(a, b):
    return jax.lax.dot_general(a, b, (((1,), (0,)), ((), ())),
                               preferred_element_type=jnp.float32)


def _dot_t(a, b):
    return jax.lax.dot_general(a, b, (((1,), (1,)), ((), ())),
                               preferred_element_type=jnp.float32)


def _log_sigmoid(x):
    return jnp.minimum(x, 0.0) - jnp.log1p(jnp.exp(-jnp.abs(x)))


def _encode_block(ns_emb, adjm, cond, Wh, Wq, Wk, Wv, Wo, Wmu, Wvar):
    s = _leaky(_dot(ns_emb, Wh))
    hidden = _leaky(_dot(adjm, s))
    q = _dot(hidden, Wq)
    k = _dot(cond, Wk)
    v = _dot(cond, Wv)
    outs = []
    for h in range(2):
        sl = slice(h * D_K, (h + 1) * D_K)
        logits = _dot_t(q[:, sl], k[:, sl]) * 0.125
        mx = jnp.max(logits, axis=1, keepdims=True)
        e = jnp.exp(logits - mx)
        attn = e / jnp.sum(e, axis=1, keepdims=True)
        outs.append(_dot(attn, v[:, sl]))
    o = _dot(jnp.concatenate(outs, axis=1), Wo)
    mu = _leaky(_dot(adjm, _leaky(_dot(o, Wmu))))
    lv = _leaky(_dot(adjm, _leaky(_dot(o, Wvar))))
    return mu, lv


def _dense_body(ns_emb_ref, adj_ref, adjp_ref, cond_ref, labels_ref, eps_ref,
                Whp_ref, Wqp_ref, Wkp_ref, Wvp_ref, Wop_ref, Wmup_ref, Wvarp_ref,
                Whr_ref, Wqr_ref, Wkr_ref, Wvr_ref, Wor_ref, Wmur_ref, Wvarr_ref,
                Wmap_ref,
                scores_ref, m_ref, recons_ref, kld_ref):
    ns_emb = ns_emb_ref[...]
    cond = cond_ref[...]
    mu, logvar = _encode_block(ns_emb, adj_ref[...], cond,
                               Whp_ref[...], Wqp_ref[...], Wkp_ref[...],
                               Wvp_ref[...], Wop_ref[...], Wmup_ref[...],
                               Wvarp_ref[...])
    mu_p, logvar_p = _encode_block(ns_emb, adjp_ref[...], cond,
                                   Whr_ref[...], Wqr_ref[...], Wkr_ref[...],
                                   Wvr_ref[...], Wor_ref[...], Wmur_ref[...],
                                   Wvarr_ref[...])
    z = eps_ref[...] * jnp.exp(0.5 * logvar) + mu
    S = _dot_t(z, z)
    recover_adj = jax.nn.sigmoid(S)

    row = lax.broadcasted_iota(jnp.int32, (N, N), 0)
    col = lax.broadcasted_iota(jnp.int32, (N, N), 1)
    scores_ref[...] = lax.bitcast_convert_type(
        jnp.where(col > row, recover_adj, 0.0), jnp.int32)

    m_ref[...] = _leaky(_dot(ns_emb, Wmap_ref[...]))

    labels = labels_ref[...]
    s_sum = jnp.sum(labels)
    nf = jnp.float32(N)
    pos_weight = (nf * nf - s_sum + nf) / (s_sum - nf + 0.01)
    norm = nf * nf / (nf * nf - s_sum + nf)
    bce = -(pos_weight * labels * _log_sigmoid(recover_adj)
            + (1.0 - labels) * _log_sigmoid(-recover_adj))
    recons_ref[...] = jnp.reshape(norm * jnp.mean(bce), (1, 1))

    kld = 0.5 / nf * jnp.mean(jnp.sum(
        (mu_p - mu) ** 2 / jnp.exp(logvar_p)
        + jnp.exp(logvar) / jnp.exp(logvar_p)
        - 1.0 - (logvar - logvar_p), axis=1))
    kld_ref[...] = jnp.reshape(kld, (1, 1))


def _dense_call(ns_emb, adj, adj_prior, cond, labels, eps, *weights):
    return pl.pallas_call(
        _dense_body,
        out_shape=(
            jax.ShapeDtypeStruct((N, N), jnp.int32),
            jax.ShapeDtypeStruct((N, H2), jnp.float32),
            jax.ShapeDtypeStruct((1, 1), jnp.float32),
            jax.ShapeDtypeStruct((1, 1), jnp.float32),
        ),
    )(ns_emb, adj, adj_prior, cond, labels, eps, *weights)


NC = 2
NS = 16
L = 16
TOT = N * N
HALF = TOT // NC
CHUNK = HALF // NS
NVR = CHUNK // L
HBINS = 256
BUF = 544


def _sc_topk_body(scores_hbm, outv_hbm, outi_hbm,
                  chunk_v, candb_v, candi_v, hist_v, red_v, gall_v, ghist_v,
                  pcum_v,
                  gtv_v, gti_v, tie_v, cnt_v, cntall_v,
                  tmpv_v, tmpi_v, cbufv_v, cbufi_v,
                  shist_s, sgtv_s, sgti_s, stie_s, scnt_s):
    c = lax.axis_index("c")
    s = lax.axis_index("s")
    base = c * HALF + s * CHUNK
    lane = lax.iota(jnp.int32, L)
    ones_i = jnp.ones((L,), jnp.int32)

    pltpu.sync_copy(scores_hbm.at[pl.ds(base, CHUNK)], chunk_v)

    prefix = jnp.int32(0)
    kp = jnp.int32(MAX_K)
    mcnt = jnp.int32(0)

    for rnd in range(4):
        shift = 24 - 8 * rnd

        def zero_body(i, _):
            hist_v[pl.ds(i * L, L)] = jnp.zeros((L,), jnp.int32)
            return 0
        lax.fori_loop(0, (NS * HBINS) // L, zero_body, 0, unroll=8)

        if rnd == 0:
            def scan_body(j, _):
                bits = chunk_v[pl.ds(j * L, L)]
                digit = (bits >> shift) & (HBINS - 1)
                addr = lane * HBINS + digit
                plsc.addupdate_scatter(hist_v, [addr], ones_i)
                return 0
            lax.fori_loop(0, NVR, scan_body, 0, unroll=8)
        else:
            def scan_body(j, _):
                bits = candb_v[pl.ds(j * L, L)]
                digit = (bits >> shift) & (HBINS - 1)
                addr = lane * HBINS + digit
                match = jnp.logical_and((bits >> (shift + 8)) == prefix,
                                        (j * L + lane) < mcnt)
                plsc.addupdate_scatter(hist_v, [addr], ones_i, mask=match)
                return 0
            lax.fori_loop(0, (mcnt + L - 1) >> 4, scan_body, 0)

        def red_body(g, _):
            acc = hist_v[pl.ds(g * L, L)]
            for l in range(1, NS):
                acc = acc + hist_v[pl.ds(l * HBINS + g * L, L)]
            red_v[pl.ds(g * L, L)] = acc
            return 0
        lax.fori_loop(0, HBINS // L, red_body, 0)

        pltpu.sync_copy(red_v, shist_s.at[s])
        plsc.subcore_barrier()
        pltpu.sync_copy(shist_s, gall_v)
        plsc.subcore_barrier()

        def sum_body(g, carry_tot):
            acc = gall_v[0, pl.ds(g * L, L)]
            for l in range(1, NS):
                acc = acc + gall_v[l, pl.ds(g * L, L)]
            ghist_v[pl.ds(g * L, L)] = acc
            cum = plsc.cumsum(acc)
            pcum_v[pl.ds(g * L, L)] = cum + carry_tot
            return carry_tot + cum[L - 1]
        total = lax.fori_loop(0, HBINS // L, sum_body, jnp.int32(0))

        def cnt_body(g, cnt):
            pc = pcum_v[pl.ds(g * L, L)]
            h = ghist_v[pl.ds(g * L, L)]
            cond = (total - pc + h) >= kp
            return cnt + jnp.sum(cond.astype(jnp.int32))
        bstar = lax.fori_loop(0, HBINS // L, cnt_body, jnp.int32(0),
                              unroll=4) - 1
        cumsel = total - pcum_v[pl.ds(bstar, L)][0]
        kp = kp - cumsel
        prefix = (prefix << 8) | bstar

        if rnd == 0:
            def comp_body(j, mc):
                bits = chunk_v[pl.ds(j * L, L)]
                keep = (bits >> 24) >= prefix
                gidx = base + j * L + lane
                plsc.store_compressed(candb_v.at[pl.ds(mc, L)], bits,
                                      mask=keep)
                plsc.store_compressed(candi_v.at[pl.ds(mc, L)], gidx,
                                      mask=keep)
                return mc + jnp.sum(keep.astype(jnp.int32))
            mcnt = lax.fori_loop(0, NVR, comp_body, jnp.int32(0), unroll=4)

    vstar = prefix
    vstar_vec = jnp.full((L,), 1, jnp.int32) * vstar

    def coll_body(j, carry):
        off_gt, off_tie = carry
        bits = candb_v[pl.ds(j * L, L)]
        gidx = candi_v[pl.ds(j * L, L)]
        valid = (j * L + lane) < mcnt
        gt = jnp.logical_and(valid, bits > vstar)
        tie = jnp.logical_and(valid, bits == vstar)

        @pl.when(off_gt < MAX_K)
        def _():
            plsc.store_compressed(gtv_v.at[pl.ds(off_gt, L)], bits, mask=gt)
            plsc.store_compressed(gti_v.at[pl.ds(off_gt, L)], gidx, mask=gt)

        @pl.when(off_tie < MAX_K)
        def _():
            plsc.store_compressed(tie_v.at[pl.ds(off_tie, L)], gidx, mask=tie)

        n_gt = jnp.sum(gt.astype(jnp.int32))
        n_tie = jnp.sum(tie.astype(jnp.int32))
        off_gt = jnp.where(off_gt < MAX_K, off_gt + n_gt, off_gt)
        off_tie = jnp.where(off_tie < MAX_K, off_tie + n_tie, off_tie)
        return off_gt, off_tie

    off_gt, off_tie = lax.fori_loop(0, (mcnt + L - 1) >> 4, coll_body,
                                    (jnp.int32(0), jnp.int32(0)))

    cnt_v[...] = jnp.where(lane == 0, off_gt,
                           jnp.where(lane == 1, off_tie, 0))
    pltpu.sync_copy(gtv_v, sgtv_s.at[s])
    pltpu.sync_copy(gti_v, sgti_s.at[s])
    pltpu.sync_copy(tie_v, stie_s.at[s])
    pltpu.sync_copy(cnt_v, scnt_s.at[s])
    plsc.subcore_barrier()

    @pl.when(s == 0)
    def _():
        pltpu.sync_copy(scnt_s, cntall_v)
        off = jnp.int32(0)
        for w in range(NS):
            cw = cntall_v[w, pl.ds(0, L)][0]

            @pl.when(cw > 0)
            def _():
                pltpu.sync_copy(sgtv_s.at[w], tmpv_v)
                pltpu.sync_copy(sgti_s.at[w], tmpi_v)

            def gt_body(j, off):
                valid = lane < (cw - j * L)
                plsc.store_compressed(cbufv_v.at[pl.ds(off, L)],
                                      tmpv_v[pl.ds(j * L, L)], mask=valid)
                plsc.store_compressed(cbufi_v.at[pl.ds(off, L)],
                                      tmpi_v[pl.ds(j * L, L)], mask=valid)
                return off + jnp.sum(valid.astype(jnp.int32))
            off = lax.fori_loop(0, (cw + L - 1) >> 4, gt_body, off)

        taken = jnp.int32(0)
        for w in range(NS):
            cw = cntall_v[w, pl.ds(0, L)][1]
            todo = jnp.maximum(jnp.minimum(cw, kp - taken), 0)

            @pl.when(todo > 0)
            def _():
                pltpu.sync_copy(stie_s.at[w], tmpi_v)

            def tie_body(j, carry):
                off, taken = carry
                valid = jnp.logical_and(lane < (cw - j * L),
                                        (taken + lane) < kp)
                plsc.store_compressed(cbufv_v.at[pl.ds(off, L)],
                                      vstar_vec, mask=valid)
                plsc.store_compressed(cbufi_v.at[pl.ds(off, L)],
                                      tmpi_v[pl.ds(j * L, L)], mask=valid)
                n = jnp.sum(valid.astype(jnp.int32))
                return off + n, taken + n
            off, taken = lax.fori_loop(0, (todo + L - 1) >> 4, tie_body,
                                       (off, taken))

        pltpu.sync_copy(cbufv_v.at[pl.ds(0, MAX_K)], outv_hbm.at[c])
        pltpu.sync_copy(cbufi_v.at[pl.ds(0, MAX_K)], outi_hbm.at[c])


def _sc_topk(scores_flat):
    mesh = plsc.VectorSubcoreMesh(core_axis_name="c", subcore_axis_name="s",
                                  num_cores=NC, num_subcores=NS)
    f = pl.kernel(
        _sc_topk_body,
        out_type=(
            jax.ShapeDtypeStruct((NC, MAX_K), jnp.int32),
            jax.ShapeDtypeStruct((NC, MAX_K), jnp.int32),
        ),
        mesh=mesh,
        compiler_params=pltpu.CompilerParams(needs_layout_passes=False),
        scratch_types=[
            pltpu.VMEM((CHUNK,), jnp.int32),
            pltpu.VMEM((CHUNK + L,), jnp.int32),
            pltpu.VMEM((CHUNK + L,), jnp.int32),
            pltpu.VMEM((NS * HBINS,), jnp.int32),
            pltpu.VMEM((HBINS,), jnp.int32),
            pltpu.VMEM((NS, HBINS), jnp.int32),
            pltpu.VMEM((HBINS + L,), jnp.int32),
            pltpu.VMEM((HBINS + L,), jnp.int32),
            pltpu.VMEM((BUF,), jnp.int32),
            pltpu.VMEM((BUF,), jnp.int32),
            pltpu.VMEM((BUF,), jnp.int32),
            pltpu.VMEM((L,), jnp.int32),
            pltpu.VMEM((NS, L), jnp.int32),
            pltpu.VMEM((BUF,), jnp.int32),
            pltpu.VMEM((BUF,), jnp.int32),
            pltpu.VMEM((BUF,), jnp.int32),
            pltpu.VMEM((BUF,), jnp.int32),
            pltpu.VMEM_SHARED((NS, HBINS), jnp.int32),
            pltpu.VMEM_SHARED((NS, BUF), jnp.int32),
            pltpu.VMEM_SHARED((NS, BUF), jnp.int32),
            pltpu.VMEM_SHARED((NS, BUF), jnp.int32),
            pltpu.VMEM_SHARED((NS, L), jnp.int32),
        ],
    )
    return f(scores_flat)


def _final_body(vals_ref, idx_ref, m_ref, rel_ref):
    vals = vals_ref[...]
    idxf = idx_ref[...].astype(jnp.float32)
    eye = (lax.broadcasted_iota(jnp.int32, (128, 128), 0)
           == lax.broadcasted_iota(jnp.int32, (128, 128), 1)).astype(jnp.float32)

    def _tcol(row):
        return lax.dot_general(eye, row, (((1,), (1,)), ((), ())),
                               preferred_element_type=jnp.float32)

    colv = [_tcol(vals[j:j + 1, :]) for j in range(8)]
    coli = [_tcol(idxf[j:j + 1, :]) for j in range(8)]

    p_f = lax.broadcasted_iota(jnp.int32, (MAX_K, 1), 0).astype(jnp.float32)
    fidx = jnp.zeros((MAX_K, 1), jnp.float32)
    for i in range(8):
        vi = vals[i:i + 1, :]
        ii = idxf[i:i + 1, :]
        acc = jnp.zeros((1, 128), jnp.float32)
        for j in range(8):
            ahead = jnp.logical_or(
                colv[j] > vi,
                jnp.logical_and(colv[j] == vi, coli[j] < ii))
            acc = acc + jnp.sum(ahead.astype(jnp.float32), axis=0,
                                keepdims=True)
        sel = (acc == p_f).astype(jnp.float32)
        fidx = fidx + jnp.sum(sel * ii, axis=1, keepdims=True)

    iidx = fidx.astype(jnp.int32)
    q = lax.broadcasted_iota(jnp.int32, (MAX_K, N), 1)
    oh_r = ((iidx >> 10) == q).astype(jnp.float32)
    oh_c = ((iidx & (N - 1)) == q).astype(jnp.float32)
    m = m_ref[...]
    rel_ref[...] = _dot(oh_r, m) + _dot(oh_c, m)


def _final_call(vals8, idx8, m):
    return pl.pallas_call(
        _final_body,
        out_shape=jax.ShapeDtypeStruct((MAX_K, H2), jnp.float32),
    )(vals8, idx8, m)


def kernel(ns_emb, adj, adj_prior, condition, labels, Wh_post, Wq_post,
           Wk_post, Wv_post, Wo_post, Wmu_post, Wvar_post, Wh_prior, Wq_prior,
           Wk_prior, Wv_prior, Wo_prior, Wmu_prior, Wvar_prior, Wmap):
    cond = condition[0]
    eps = jax.random.normal(jax.random.key(42), (N, H2), dtype=jnp.float32)
    scores, m, recons, kld = _dense_call(
        ns_emb, adj, adj_prior, cond, labels, eps,
        Wh_post, Wq_post, Wk_post, Wv_post, Wo_post, Wmu_post, Wvar_post,
        Wh_prior, Wq_prior, Wk_prior, Wv_prior, Wo_prior, Wmu_prior,
        Wvar_prior, Wmap)

    outv, outi = _sc_topk(scores.reshape(-1))
    vals8 = lax.bitcast_convert_type(outv, jnp.float32).reshape(8, 128)
    relations = _final_call(vals8, outi.reshape(8, 128), m)
    rel_mask = jnp.zeros((MAX_K,), dtype=jnp.bool_)
    return relations, rel_mask, recons[0, 0], kld[0, 0]


# speedup vs baseline: 7.2112x; 1.0776x over previous
import functools

import jax
import jax.numpy as jnp
from jax import lax
from jax.experimental import pallas as pl
from jax.experimental.pallas import tpu as pltpu
from jax.experimental.pallas import tpu_sc as plsc

N = 1024
IN_DIM = 256
H1 = 128
H2 = 32
COND_LEN = 64
D_K = 64
MAX_K = 512


def _leaky(x):
    return jnp.where(x >= 0, x, 0.01 * x)


def _dot(a, b):
    return jax.lax.dot_general(a, b, (((1,), (0,)), ((), ())),
                               preferred_element_type=jnp.float32)


def _dot_t(a, b):
    return jax.lax.dot_general(a, b, (((1,), (1,)), ((), ())),
                               preferred_element_type=jnp.float32)


def _log_sigmoid(x):
    return jnp.minimum(x, 0.0) - jnp.log1p(jnp.exp(-jnp.abs(x)))


def _encode_block(ns_emb, adjm, cond, Wh, Wq, Wk, Wv, Wo, Wmu, Wvar):
    s = _leaky(_dot(ns_emb, Wh))
    hidden = _leaky(_dot(adjm, s))
    q = _dot(hidden, Wq)
    k = _dot(cond, Wk)
    v = _dot(cond, Wv)
    outs = []
    for h in range(2):
        sl = slice(h * D_K, (h + 1) * D_K)
        logits = _dot_t(q[:, sl], k[:, sl]) * 0.125
        mx = jnp.max(logits, axis=1, keepdims=True)
        e = jnp.exp(logits - mx)
        attn = e / jnp.sum(e, axis=1, keepdims=True)
        outs.append(_dot(attn, v[:, sl]))
    o = _dot(jnp.concatenate(outs, axis=1), Wo)
    mu = _leaky(_dot(adjm, _leaky(_dot(o, Wmu))))
    lv = _leaky(_dot(adjm, _leaky(_dot(o, Wvar))))
    return mu, lv


def _dense_body(ns_emb_ref, adj_ref, adjp_ref, cond_ref, labels_ref, eps_ref,
                Whp_ref, Wqp_ref, Wkp_ref, Wvp_ref, Wop_ref, Wmup_ref, Wvarp_ref,
                Whr_ref, Wqr_ref, Wkr_ref, Wvr_ref, Wor_ref, Wmur_ref, Wvarr_ref,
                Wmap_ref,
                scores_ref, m_ref, recons_ref, kld_ref):
    ns_emb = ns_emb_ref[...]
    cond = cond_ref[...]
    mu, logvar = _encode_block(ns_emb, adj_ref[...], cond,
                               Whp_ref[...], Wqp_ref[...], Wkp_ref[...],
                               Wvp_ref[...], Wop_ref[...], Wmup_ref[...],
                               Wvarp_ref[...])
    mu_p, logvar_p = _encode_block(ns_emb, adjp_ref[...], cond,
                                   Whr_ref[...], Wqr_ref[...], Wkr_ref[...],
                                   Wvr_ref[...], Wor_ref[...], Wmur_ref[...],
                                   Wvarr_ref[...])
    z = eps_ref[...] * jnp.exp(0.5 * logvar) + mu
    S = _dot_t(z, z)
    recover_adj = jax.nn.sigmoid(S)

    row = lax.broadcasted_iota(jnp.int32, (N, N), 0)
    col = lax.broadcasted_iota(jnp.int32, (N, N), 1)
    scores_ref[...] = lax.bitcast_convert_type(
        jnp.where(col > row, recover_adj, 0.0), jnp.int32)

    m_ref[...] = _leaky(_dot(ns_emb, Wmap_ref[...]))

    labels = labels_ref[...]
    s_sum = jnp.sum(labels)
    nf = jnp.float32(N)
    pos_weight = (nf * nf - s_sum + nf) / (s_sum - nf + 0.01)
    norm = nf * nf / (nf * nf - s_sum + nf)
    bce = -(pos_weight * labels * _log_sigmoid(recover_adj)
            + (1.0 - labels) * _log_sigmoid(-recover_adj))
    recons_ref[...] = jnp.reshape(norm * jnp.mean(bce), (1, 1))

    kld = 0.5 / nf * jnp.mean(jnp.sum(
        (mu_p - mu) ** 2 / jnp.exp(logvar_p)
        + jnp.exp(logvar) / jnp.exp(logvar_p)
        - 1.0 - (logvar - logvar_p), axis=1))
    kld_ref[...] = jnp.reshape(kld, (1, 1))


def _dense_call(ns_emb, adj, adj_prior, cond, labels, eps, *weights):
    return pl.pallas_call(
        _dense_body,
        out_shape=(
            jax.ShapeDtypeStruct((N, N), jnp.int32),
            jax.ShapeDtypeStruct((N, H2), jnp.float32),
            jax.ShapeDtypeStruct((1, 1), jnp.float32),
            jax.ShapeDtypeStruct((1, 1), jnp.float32),
        ),
    )(ns_emb, adj, adj_prior, cond, labels, eps, *weights)


NC = 2
NS = 16
L = 16
TOT = N * N
HALF = TOT // NC
CHUNK = HALF // NS
NVR = CHUNK // L
HBINS = 256
BUF = 544
U = 2


def _sc_topk_body(scores_hbm, outv_hbm, outi_hbm,
                  chunk_v, candb_v, candi_v, hist_v, red_v, gall_v, ghist_v,
                  pcum_v,
                  gtv_v, gti_v, tie_v, cnt_v, cntall_v,
                  tmpv_v, tmpi_v, cbufv_v, cbufi_v,
                  shist_s, sgtv_s, sgti_s, stie_s, scnt_s):
    c = lax.axis_index("c")
    s = lax.axis_index("s")
    base = c * HALF + s * CHUNK
    lane = lax.iota(jnp.int32, L)
    ones_i = jnp.ones((L,), jnp.int32)

    pltpu.sync_copy(scores_hbm.at[pl.ds(base, CHUNK)], chunk_v)

    prefix = jnp.int32(0)
    kp = jnp.int32(MAX_K)
    mcnt = jnp.int32(0)

    for rnd in range(4):
        shift = 24 - 8 * rnd

        nzero = (U * NS * HBINS) // L if rnd == 0 else (NS * HBINS) // L

        @plsc.parallel_loop(0, nzero, unroll=8)
        def zero_body(i):
            hist_v[pl.ds(i * L, L)] = jnp.zeros((L,), jnp.int32)

        if rnd == 0:
            @plsc.parallel_loop(0, NVR, unroll=8)
            def scan_body(j):
                bits = chunk_v[pl.ds(j * L, L)]
                digit = (bits >> shift) & (HBINS - 1)
                addr = (j & (U - 1)) * (NS * HBINS) + lane * HBINS + digit
                plsc.addupdate_scatter(hist_v, [addr], ones_i)
        else:
            def scan_body(j, _):
                bits = candb_v[pl.ds(j * L, L)]
                digit = (bits >> shift) & (HBINS - 1)
                addr = lane * HBINS + digit
                match = jnp.logical_and((bits >> (shift + 8)) == prefix,
                                        (j * L + lane) < mcnt)
                plsc.addupdate_scatter(hist_v, [addr], ones_i, mask=match)
                return 0
            lax.fori_loop(0, (mcnt + L - 1) >> 4, scan_body, 0)

        ncopy = U * NS if rnd == 0 else NS

        @plsc.parallel_loop(0, HBINS // L, unroll=2)
        def red_body(g):
            acc = hist_v[pl.ds(g * L, L)]
            for l in range(1, ncopy):
                acc = acc + hist_v[pl.ds(l * HBINS + g * L, L)]
            red_v[pl.ds(g * L, L)] = acc

        pltpu.sync_copy(red_v, shist_s.at[s])
        plsc.subcore_barrier()
        pltpu.sync_copy(shist_s, gall_v)
        plsc.subcore_barrier()

        def sum_body(g, carry_tot):
            acc = gall_v[0, pl.ds(g * L, L)]
            for l in range(1, NS):
                acc = acc + gall_v[l, pl.ds(g * L, L)]
            ghist_v[pl.ds(g * L, L)] = acc
            cum = plsc.cumsum(acc)
            pcum_v[pl.ds(g * L, L)] = cum + carry_tot
            return carry_tot + cum[L - 1]
        total = lax.fori_loop(0, HBINS // L, sum_body, jnp.int32(0))

        def cnt_body(g, cnt):
            pc = pcum_v[pl.ds(g * L, L)]
            h = ghist_v[pl.ds(g * L, L)]
            cond = (total - pc + h) >= kp
            return cnt + jnp.sum(cond.astype(jnp.int32))
        bstar = lax.fori_loop(0, HBINS // L, cnt_body, jnp.int32(0),
                              unroll=4) - 1
        cumsel = total - pcum_v[pl.ds(bstar, L)][0]
        kp = kp - cumsel
        prefix = (prefix << 8) | bstar

        if rnd == 0:
            def comp_body(j, mc):
                bits = chunk_v[pl.ds(j * L, L)]
                keep = (bits >> 24) >= prefix
                gidx = base + j * L + lane
                plsc.store_compressed(candb_v.at[pl.ds(mc, L)], bits,
                                      mask=keep)
                plsc.store_compressed(candi_v.at[pl.ds(mc, L)], gidx,
                                      mask=keep)
                return mc + jnp.sum(keep.astype(jnp.int32))
            mcnt = lax.fori_loop(0, NVR, comp_body, jnp.int32(0), unroll=4)

    vstar = prefix
    vstar_vec = jnp.full((L,), 1, jnp.int32) * vstar

    def coll_body(j, carry):
        off_gt, off_tie = carry
        bits = candb_v[pl.ds(j * L, L)]
        gidx = candi_v[pl.ds(j * L, L)]
        valid = (j * L + lane) < mcnt
        gt = jnp.logical_and(valid, bits > vstar)
        tie = jnp.logical_and(valid, bits == vstar)

        @pl.when(off_gt < MAX_K)
        def _():
            plsc.store_compressed(gtv_v.at[pl.ds(off_gt, L)], bits, mask=gt)
            plsc.store_compressed(gti_v.at[pl.ds(off_gt, L)], gidx, mask=gt)

        @pl.when(off_tie < MAX_K)
        def _():
            plsc.store_compressed(tie_v.at[pl.ds(off_tie, L)], gidx, mask=tie)

        n_gt = jnp.sum(gt.astype(jnp.int32))
        n_tie = jnp.sum(tie.astype(jnp.int32))
        off_gt = jnp.where(off_gt < MAX_K, off_gt + n_gt, off_gt)
        off_tie = jnp.where(off_tie < MAX_K, off_tie + n_tie, off_tie)
        return off_gt, off_tie

    off_gt, off_tie = lax.fori_loop(0, (mcnt + L - 1) >> 4, coll_body,
                                    (jnp.int32(0), jnp.int32(0)))

    cnt_v[...] = jnp.where(lane == 0, off_gt,
                           jnp.where(lane == 1, off_tie, 0))
    pltpu.sync_copy(gtv_v, sgtv_s.at[s])
    pltpu.sync_copy(gti_v, sgti_s.at[s])
    pltpu.sync_copy(tie_v, stie_s.at[s])
    pltpu.sync_copy(cnt_v, scnt_s.at[s])
    plsc.subcore_barrier()

    @pl.when(s == 0)
    def _():
        pltpu.sync_copy(scnt_s, cntall_v)
        off = jnp.int32(0)
        for w in range(NS):
            cw = cntall_v[w, pl.ds(0, L)][0]

            @pl.when(cw > 0)
            def _():
                pltpu.sync_copy(sgtv_s.at[w], tmpv_v)
                pltpu.sync_copy(sgti_s.at[w], tmpi_v)

            def gt_body(j, off):
                valid = lane < (cw - j * L)
                plsc.store_compressed(cbufv_v.at[pl.ds(off, L)],
                                      tmpv_v[pl.ds(j * L, L)], mask=valid)
                plsc.store_compressed(cbufi_v.at[pl.ds(off, L)],
                                      tmpi_v[pl.ds(j * L, L)], mask=valid)
                return off + jnp.sum(valid.astype(jnp.int32))
            off = lax.fori_loop(0, (cw + L - 1) >> 4, gt_body, off)

        taken = jnp.int32(0)
        for w in range(NS):
            cw = cntall_v[w, pl.ds(0, L)][1]
            todo = jnp.maximum(jnp.minimum(cw, kp - taken), 0)

            @pl.when(todo > 0)
            def _():
                pltpu.sync_copy(stie_s.at[w], tmpi_v)

            def tie_body(j, carry):
                off, taken = carry
                valid = jnp.logical_and(lane < (cw - j * L),
                                        (taken + lane) < kp)
                plsc.store_compressed(cbufv_v.at[pl.ds(off, L)],
                                      vstar_vec, mask=valid)
                plsc.store_compressed(cbufi_v.at[pl.ds(off, L)],
                                      tmpi_v[pl.ds(j * L, L)], mask=valid)
                n = jnp.sum(valid.astype(jnp.int32))
                return off + n, taken + n
            off, taken = lax.fori_loop(0, (todo + L - 1) >> 4, tie_body,
                                       (off, taken))

        pltpu.sync_copy(cbufv_v.at[pl.ds(0, MAX_K)], outv_hbm.at[c])
        pltpu.sync_copy(cbufi_v.at[pl.ds(0, MAX_K)], outi_hbm.at[c])


def _sc_topk(scores_flat):
    mesh = plsc.VectorSubcoreMesh(core_axis_name="c", subcore_axis_name="s",
                                  num_cores=NC, num_subcores=NS)
    f = pl.kernel(
        _sc_topk_body,
        out_type=(
            jax.ShapeDtypeStruct((NC, MAX_K), jnp.int32),
            jax.ShapeDtypeStruct((NC, MAX_K), jnp.int32),
        ),
        mesh=mesh,
        compiler_params=pltpu.CompilerParams(needs_layout_passes=False),
        scratch_types=[
            pltpu.VMEM((CHUNK,), jnp.int32),
            pltpu.VMEM((CHUNK + L,), jnp.int32),
            pltpu.VMEM((CHUNK + L,), jnp.int32),
            pltpu.VMEM((U * NS * HBINS,), jnp.int32),
            pltpu.VMEM((HBINS,), jnp.int32),
            pltpu.VMEM((NS, HBINS), jnp.int32),
            pltpu.VMEM((HBINS + L,), jnp.int32),
            pltpu.VMEM((HBINS + L,), jnp.int32),
            pltpu.VMEM((BUF,), jnp.int32),
            pltpu.VMEM((BUF,), jnp.int32),
            pltpu.VMEM((BUF,), jnp.int32),
            pltpu.VMEM((L,), jnp.int32),
            pltpu.VMEM((NS, L), jnp.int32),
            pltpu.VMEM((BUF,), jnp.int32),
            pltpu.VMEM((BUF,), jnp.int32),
            pltpu.VMEM((BUF,), jnp.int32),
            pltpu.VMEM((BUF,), jnp.int32),
            pltpu.VMEM_SHARED((NS, HBINS), jnp.int32),
            pltpu.VMEM_SHARED((NS, BUF), jnp.int32),
            pltpu.VMEM_SHARED((NS, BUF), jnp.int32),
            pltpu.VMEM_SHARED((NS, BUF), jnp.int32),
            pltpu.VMEM_SHARED((NS, L), jnp.int32),
        ],
    )
    return f(scores_flat)


def _final_body(vals_ref, idx_ref, m_ref, rel_ref):
    vals = vals_ref[...]
    idxf = idx_ref[...].astype(jnp.float32)
    eye = (lax.broadcasted_iota(jnp.int32, (128, 128), 0)
           == lax.broadcasted_iota(jnp.int32, (128, 128), 1)).astype(jnp.float32)

    def _tcol(row):
        return lax.dot_general(eye, row, (((1,), (1,)), ((), ())),
                               preferred_element_type=jnp.float32)

    colv = [_tcol(vals[j:j + 1, :]) for j in range(8)]
    coli = [_tcol(idxf[j:j + 1, :]) for j in range(8)]

    p_f = lax.broadcasted_iota(jnp.int32, (MAX_K, 1), 0).astype(jnp.float32)
    fidx = jnp.zeros((MAX_K, 1), jnp.float32)
    for i in range(8):
        vi = vals[i:i + 1, :]
        ii = idxf[i:i + 1, :]
        acc = jnp.zeros((1, 128), jnp.float32)
        for j in range(8):
            ahead = jnp.logical_or(
                colv[j] > vi,
                jnp.logical_and(colv[j] == vi, coli[j] < ii))
            acc = acc + jnp.sum(ahead.astype(jnp.float32), axis=0,
                                keepdims=True)
        sel = (acc == p_f).astype(jnp.float32)
        fidx = fidx + jnp.sum(sel * ii, axis=1, keepdims=True)

    iidx = fidx.astype(jnp.int32)
    q = lax.broadcasted_iota(jnp.int32, (MAX_K, N), 1)
    oh_r = ((iidx >> 10) == q).astype(jnp.float32)
    oh_c = ((iidx & (N - 1)) == q).astype(jnp.float32)
    m = m_ref[...]
    rel_ref[...] = _dot(oh_r, m) + _dot(oh_c, m)


def _final_call(vals8, idx8, m):
    return pl.pallas_call(
        _final_body,
        out_shape=jax.ShapeDtypeStruct((MAX_K, H2), jnp.float32),
    )(vals8, idx8, m)


def kernel(ns_emb, adj, adj_prior, condition, labels, Wh_post, Wq_post,
           Wk_post, Wv_post, Wo_post, Wmu_post, Wvar_post, Wh_prior, Wq_prior,
           Wk_prior, Wv_prior, Wo_prior, Wmu_prior, Wvar_prior, Wmap):
    cond = condition[0]
    eps = jax.random.normal(jax.random.key(42), (N, H2), dtype=jnp.float32)
    scores, m, recons, kld = _dense_call(
        ns_emb, adj, adj_prior, cond, labels, eps,
        Wh_post, Wq_post, Wk_post, Wv_post, Wo_post, Wmu_post, Wvar_post,
        Wh_prior, Wq_prior, Wk_prior, Wv_prior, Wo_prior, Wmu_prior,
        Wvar_prior, Wmap)

    outv, outi = _sc_topk(scores.reshape(-1))
    vals8 = lax.bitcast_convert_type(outv, jnp.float32).reshape(8, 128)
    relations = _final_call(vals8, outi.reshape(8, 128), m)
    rel_mask = jnp.zeros((MAX_K,), dtype=jnp.bool_)
    return relations, rel_mask, recons[0, 0], kld[0, 0]


# speedup vs baseline: 8.6577x; 1.2006x over previous
import functools

import jax
import jax.numpy as jnp
from jax import lax
from jax.experimental import pallas as pl
from jax.experimental.pallas import tpu as pltpu
from jax.experimental.pallas import tpu_sc as plsc

N = 1024
IN_DIM = 256
H1 = 128
H2 = 32
COND_LEN = 64
D_K = 64
MAX_K = 512


def _leaky(x):
    return jnp.where(x >= 0, x, 0.01 * x)


def _dot(a, b):
    return jax.lax.dot_general(a, b, (((1,), (0,)), ((), ())),
                               preferred_element_type=jnp.float32)


def _dot_t(a, b):
    return jax.lax.dot_general(a, b, (((1,), (1,)), ((), ())),
                               preferred_element_type=jnp.float32)


def _log_sigmoid(x):
    return jnp.minimum(x, 0.0) - jnp.log1p(jnp.exp(-jnp.abs(x)))


def _encode_block(ns_emb, adjm, cond, Wh, Wq, Wk, Wv, Wo, Wmu, Wvar):
    s = _leaky(_dot(ns_emb, Wh))
    hidden = _leaky(_dot(adjm, s))
    q = _dot(hidden, Wq)
    k = _dot(cond, Wk)
    v = _dot(cond, Wv)
    outs = []
    for h in range(2):
        sl = slice(h * D_K, (h + 1) * D_K)
        logits = _dot_t(q[:, sl], k[:, sl]) * 0.125
        mx = jnp.max(logits, axis=1, keepdims=True)
        e = jnp.exp(logits - mx)
        attn = e / jnp.sum(e, axis=1, keepdims=True)
        outs.append(_dot(attn, v[:, sl]))
    o = _dot(jnp.concatenate(outs, axis=1), Wo)
    mu = _leaky(_dot(adjm, _leaky(_dot(o, Wmu))))
    lv = _leaky(_dot(adjm, _leaky(_dot(o, Wvar))))
    return mu, lv


def _dense_body(ns_emb_ref, adj_ref, adjp_ref, cond_ref, labels_ref, eps_ref,
                Whp_ref, Wqp_ref, Wkp_ref, Wvp_ref, Wop_ref, Wmup_ref, Wvarp_ref,
                Whr_ref, Wqr_ref, Wkr_ref, Wvr_ref, Wor_ref, Wmur_ref, Wvarr_ref,
                Wmap_ref,
                scores_ref, m_ref, recons_ref, kld_ref):
    ns_emb = ns_emb_ref[...]
    cond = cond_ref[...]
    mu, logvar = _encode_block(ns_emb, adj_ref[...], cond,
                               Whp_ref[...], Wqp_ref[...], Wkp_ref[...],
                               Wvp_ref[...], Wop_ref[...], Wmup_ref[...],
                               Wvarp_ref[...])
    mu_p, logvar_p = _encode_block(ns_emb, adjp_ref[...], cond,
                                   Whr_ref[...], Wqr_ref[...], Wkr_ref[...],
                                   Wvr_ref[...], Wor_ref[...], Wmur_ref[...],
                                   Wvarr_ref[...])
    z = eps_ref[...] * jnp.exp(0.5 * logvar) + mu
    S = _dot_t(z, z)
    recover_adj = jax.nn.sigmoid(S)

    row = lax.broadcasted_iota(jnp.int32, (N, N), 0)
    col = lax.broadcasted_iota(jnp.int32, (N, N), 1)
    scores_ref[...] = lax.bitcast_convert_type(
        jnp.where(col > row, recover_adj, 0.0), jnp.int32)

    m_ref[...] = _leaky(_dot(ns_emb, Wmap_ref[...]))

    labels = labels_ref[...]
    s_sum = jnp.sum(labels)
    nf = jnp.float32(N)
    pos_weight = (nf * nf - s_sum + nf) / (s_sum - nf + 0.01)
    norm = nf * nf / (nf * nf - s_sum + nf)
    bce = -(pos_weight * labels * _log_sigmoid(recover_adj)
            + (1.0 - labels) * _log_sigmoid(-recover_adj))
    recons_ref[...] = jnp.reshape(norm * jnp.mean(bce), (1, 1))

    kld = 0.5 / nf * jnp.mean(jnp.sum(
        (mu_p - mu) ** 2 / jnp.exp(logvar_p)
        + jnp.exp(logvar) / jnp.exp(logvar_p)
        - 1.0 - (logvar - logvar_p), axis=1))
    kld_ref[...] = jnp.reshape(kld, (1, 1))


def _dense_call(ns_emb, adj, adj_prior, cond, labels, eps, *weights):
    return pl.pallas_call(
        _dense_body,
        out_shape=(
            jax.ShapeDtypeStruct((N, N), jnp.int32),
            jax.ShapeDtypeStruct((N, H2), jnp.float32),
            jax.ShapeDtypeStruct((1, 1), jnp.float32),
            jax.ShapeDtypeStruct((1, 1), jnp.float32),
        ),
    )(ns_emb, adj, adj_prior, cond, labels, eps, *weights)


NC = 2
NS = 16
L = 16
TOT = N * N
HALF = TOT // NC
CHUNK = HALF // NS
NVR = CHUNK // L
HBINS = 256
BUF = 544
U = 2
NCH = 4
RSZ = NVR // NCH
RCAP = CHUNK // NCH


def _sc_topk_body(scores_hbm, outv_hbm, outi_hbm,
                  chunk_v, candb_v, candi_v, hist_v, red_v, gall_v, ghist_v,
                  pcum_v,
                  gtv_v, gti_v, tie_v, cnt_v, cntall_v,
                  tmpv_v, tmpi_v, cbufv_v, cbufi_v,
                  shist_s, sgtv_s, sgti_s, stie_s, scnt_s):
    c = lax.axis_index("c")
    s = lax.axis_index("s")
    base = c * HALF + s * CHUNK
    lane = lax.iota(jnp.int32, L)
    ones_i = jnp.ones((L,), jnp.int32)

    pltpu.sync_copy(scores_hbm.at[pl.ds(base, CHUNK)], chunk_v)

    prefix = jnp.int32(0)
    kp = jnp.int32(MAX_K)
    mcs = tuple(jnp.int32(0) for _ in range(NCH))

    for rnd in range(4):
        shift = 24 - 8 * rnd

        @plsc.parallel_loop(0, (U * NS * HBINS) // L, unroll=8)
        def zero_body(i):
            hist_v[pl.ds(i * L, L)] = jnp.zeros((L,), jnp.int32)

        if rnd == 0:
            @plsc.parallel_loop(0, NVR, unroll=8)
            def scan_body(j):
                bits = chunk_v[pl.ds(j * L, L)]
                digit = (bits >> shift) & (HBINS - 1)
                addr = (j & (U - 1)) * (NS * HBINS) + lane * HBINS + digit
                plsc.addupdate_scatter(hist_v, [addr], ones_i)
        else:
            for r in range(NCH):
                mc = mcs[r]
                rbase = r * (RCAP + L)

                @plsc.parallel_loop(0, (mc + L - 1) >> 4, unroll=4)
                def scan_body(j):
                    bits = candb_v[pl.ds(rbase + j * L, L)]
                    digit = (bits >> shift) & (HBINS - 1)
                    addr = ((j & (U - 1)) * (NS * HBINS)
                            + lane * HBINS + digit)
                    match = jnp.logical_and(
                        (bits >> (shift + 8)) == prefix,
                        (j * L + lane) < mc)
                    plsc.addupdate_scatter(hist_v, [addr], ones_i,
                                           mask=match)

        @plsc.parallel_loop(0, HBINS // L, unroll=2)
        def red_body(g):
            acc = hist_v[pl.ds(g * L, L)]
            for l in range(1, U * NS):
                acc = acc + hist_v[pl.ds(l * HBINS + g * L, L)]
            red_v[pl.ds(g * L, L)] = acc

        pltpu.sync_copy(red_v, shist_s.at[s])
        plsc.subcore_barrier()
        pltpu.sync_copy(shist_s, gall_v)
        plsc.subcore_barrier()

        def sum_body(g, carry_tot):
            acc = gall_v[0, pl.ds(g * L, L)]
            for l in range(1, NS):
                acc = acc + gall_v[l, pl.ds(g * L, L)]
            ghist_v[pl.ds(g * L, L)] = acc
            cum = plsc.cumsum(acc)
            pcum_v[pl.ds(g * L, L)] = cum + carry_tot
            return carry_tot + cum[L - 1]
        total = lax.fori_loop(0, HBINS // L, sum_body, jnp.int32(0))

        def cnt_body(g, cnt):
            pc = pcum_v[pl.ds(g * L, L)]
            h = ghist_v[pl.ds(g * L, L)]
            cond = (total - pc + h) >= kp
            return cnt + jnp.sum(cond.astype(jnp.int32))
        bstar = lax.fori_loop(0, HBINS // L, cnt_body, jnp.int32(0),
                              unroll=4) - 1
        cumsel = total - pcum_v[pl.ds(bstar, L)][0]
        kp = kp - cumsel
        prefix = (prefix << 8) | bstar

        if rnd == 0:
            def comp_body(j, mcs):
                new = []
                for r in range(NCH):
                    jj = r * RSZ + j
                    bits = chunk_v[pl.ds(jj * L, L)]
                    keep = (bits >> 24) >= prefix
                    gidx = base + jj * L + lane
                    dst = r * (RCAP + L) + mcs[r]
                    plsc.store_compressed(candb_v.at[pl.ds(dst, L)], bits,
                                          mask=keep)
                    plsc.store_compressed(candi_v.at[pl.ds(dst, L)], gidx,
                                          mask=keep)
                    new.append(mcs[r] + jnp.sum(keep.astype(jnp.int32)))
                return tuple(new)
            mcs = lax.fori_loop(0, RSZ, comp_body, mcs, unroll=2)

    vstar = prefix
    vstar_vec = jnp.full((L,), 1, jnp.int32) * vstar

    carry = (jnp.int32(0), jnp.int32(0))
    for r in range(NCH):
        mc = mcs[r]
        rbase = r * (RCAP + L)

        def coll_body(j, carry):
            off_gt, off_tie = carry
            bits = candb_v[pl.ds(rbase + j * L, L)]
            gidx = candi_v[pl.ds(rbase + j * L, L)]
            valid = (j * L + lane) < mc
            gt = jnp.logical_and(valid, bits > vstar)
            tie = jnp.logical_and(valid, bits == vstar)

            @pl.when(off_gt < MAX_K)
            def _():
                plsc.store_compressed(gtv_v.at[pl.ds(off_gt, L)], bits,
                                      mask=gt)
                plsc.store_compressed(gti_v.at[pl.ds(off_gt, L)], gidx,
                                      mask=gt)

            @pl.when(off_tie < MAX_K)
            def _():
                plsc.store_compressed(tie_v.at[pl.ds(off_tie, L)], gidx,
                                      mask=tie)

            n_gt = jnp.sum(gt.astype(jnp.int32))
            n_tie = jnp.sum(tie.astype(jnp.int32))
            off_gt = jnp.where(off_gt < MAX_K, off_gt + n_gt, off_gt)
            off_tie = jnp.where(off_tie < MAX_K, off_tie + n_tie, off_tie)
            return off_gt, off_tie

        carry = lax.fori_loop(0, (mc + L - 1) >> 4, coll_body, carry)
    off_gt, off_tie = carry

    cnt_v[...] = jnp.where(lane == 0, off_gt,
                           jnp.where(lane == 1, off_tie, 0))
    pltpu.sync_copy(gtv_v, sgtv_s.at[s])
    pltpu.sync_copy(gti_v, sgti_s.at[s])
    pltpu.sync_copy(tie_v, stie_s.at[s])
    pltpu.sync_copy(cnt_v, scnt_s.at[s])
    plsc.subcore_barrier()

    @pl.when(s == 0)
    def _():
        pltpu.sync_copy(scnt_s, cntall_v)
        off = jnp.int32(0)
        for w in range(NS):
            cw = cntall_v[w, pl.ds(0, L)][0]

            @pl.when(cw > 0)
            def _():
                pltpu.sync_copy(sgtv_s.at[w], tmpv_v)
                pltpu.sync_copy(sgti_s.at[w], tmpi_v)

            def gt_body(j, off):
                valid = lane < (cw - j * L)
                plsc.store_compressed(cbufv_v.at[pl.ds(off, L)],
                                      tmpv_v[pl.ds(j * L, L)], mask=valid)
                plsc.store_compressed(cbufi_v.at[pl.ds(off, L)],
                                      tmpi_v[pl.ds(j * L, L)], mask=valid)
                return off + jnp.sum(valid.astype(jnp.int32))
            off = lax.fori_loop(0, (cw + L - 1) >> 4, gt_body, off)

        taken = jnp.int32(0)
        for w in range(NS):
            cw = cntall_v[w, pl.ds(0, L)][1]
            todo = jnp.maximum(jnp.minimum(cw, kp - taken), 0)

            @pl.when(todo > 0)
            def _():
                pltpu.sync_copy(stie_s.at[w], tmpi_v)

            def tie_body(j, carry):
                off, taken = carry
                valid = jnp.logical_and(lane < (cw - j * L),
                                        (taken + lane) < kp)
                plsc.store_compressed(cbufv_v.at[pl.ds(off, L)],
                                      vstar_vec, mask=valid)
                plsc.store_compressed(cbufi_v.at[pl.ds(off, L)],
                                      tmpi_v[pl.ds(j * L, L)], mask=valid)
                n = jnp.sum(valid.astype(jnp.int32))
                return off + n, taken + n
            off, taken = lax.fori_loop(0, (todo + L - 1) >> 4, tie_body,
                                       (off, taken))

        pltpu.sync_copy(cbufv_v.at[pl.ds(0, MAX_K)], outv_hbm.at[c])
        pltpu.sync_copy(cbufi_v.at[pl.ds(0, MAX_K)], outi_hbm.at[c])


def _sc_topk(scores_flat):
    mesh = plsc.VectorSubcoreMesh(core_axis_name="c", subcore_axis_name="s",
                                  num_cores=NC, num_subcores=NS)
    f = pl.kernel(
        _sc_topk_body,
        out_type=(
            jax.ShapeDtypeStruct((NC, MAX_K), jnp.int32),
            jax.ShapeDtypeStruct((NC, MAX_K), jnp.int32),
        ),
        mesh=mesh,
        compiler_params=pltpu.CompilerParams(needs_layout_passes=False),
        scratch_types=[
            pltpu.VMEM((CHUNK,), jnp.int32),
            pltpu.VMEM((CHUNK + NCH * L,), jnp.int32),
            pltpu.VMEM((CHUNK + NCH * L,), jnp.int32),
            pltpu.VMEM((U * NS * HBINS,), jnp.int32),
            pltpu.VMEM((HBINS,), jnp.int32),
            pltpu.VMEM((NS, HBINS), jnp.int32),
            pltpu.VMEM((HBINS + L,), jnp.int32),
            pltpu.VMEM((HBINS + L,), jnp.int32),
            pltpu.VMEM((BUF,), jnp.int32),
            pltpu.VMEM((BUF,), jnp.int32),
            pltpu.VMEM((BUF,), jnp.int32),
            pltpu.VMEM((L,), jnp.int32),
            pltpu.VMEM((NS, L), jnp.int32),
            pltpu.VMEM((BUF,), jnp.int32),
            pltpu.VMEM((BUF,), jnp.int32),
            pltpu.VMEM((BUF,), jnp.int32),
            pltpu.VMEM((BUF,), jnp.int32),
            pltpu.VMEM_SHARED((NS, HBINS), jnp.int32),
            pltpu.VMEM_SHARED((NS, BUF), jnp.int32),
            pltpu.VMEM_SHARED((NS, BUF), jnp.int32),
            pltpu.VMEM_SHARED((NS, BUF), jnp.int32),
            pltpu.VMEM_SHARED((NS, L), jnp.int32),
        ],
    )
    return f(scores_flat)


def _final_body(vals_ref, idx_ref, m_ref, rel_ref):
    vals = vals_ref[...]
    idxf = idx_ref[...].astype(jnp.float32)
    eye = (lax.broadcasted_iota(jnp.int32, (128, 128), 0)
           == lax.broadcasted_iota(jnp.int32, (128, 128), 1)).astype(jnp.float32)

    def _tcol(row):
        return lax.dot_general(eye, row, (((1,), (1,)), ((), ())),
                               preferred_element_type=jnp.float32)

    colv = [_tcol(vals[j:j + 1, :]) for j in range(8)]
    coli = [_tcol(idxf[j:j + 1, :]) for j in range(8)]

    p_f = lax.broadcasted_iota(jnp.int32, (MAX_K, 1), 0).astype(jnp.float32)
    fidx = jnp.zeros((MAX_K, 1), jnp.float32)
    for i in range(8):
        vi = vals[i:i + 1, :]
        ii = idxf[i:i + 1, :]
        acc = jnp.zeros((1, 128), jnp.float32)
        for j in range(8):
            ahead = jnp.logical_or(
                colv[j] > vi,
                jnp.logical_and(colv[j] == vi, coli[j] < ii))
            acc = acc + jnp.sum(ahead.astype(jnp.float32), axis=0,
                                keepdims=True)
        sel = (acc == p_f).astype(jnp.float32)
        fidx = fidx + jnp.sum(sel * ii, axis=1, keepdims=True)

    iidx = fidx.astype(jnp.int32)
    q = lax.broadcasted_iota(jnp.int32, (MAX_K, N), 1)
    oh_r = ((iidx >> 10) == q).astype(jnp.float32)
    oh_c = ((iidx & (N - 1)) == q).astype(jnp.float32)
    m = m_ref[...]
    rel_ref[...] = _dot(oh_r, m) + _dot(oh_c, m)


def _final_call(vals8, idx8, m):
    return pl.pallas_call(
        _final_body,
        out_shape=jax.ShapeDtypeStruct((MAX_K, H2), jnp.float32),
    )(vals8, idx8, m)


def kernel(ns_emb, adj, adj_prior, condition, labels, Wh_post, Wq_post,
           Wk_post, Wv_post, Wo_post, Wmu_post, Wvar_post, Wh_prior, Wq_prior,
           Wk_prior, Wv_prior, Wo_prior, Wmu_prior, Wvar_prior, Wmap):
    cond = condition[0]
    eps = jax.random.normal(jax.random.key(42), (N, H2), dtype=jnp.float32)
    scores, m, recons, kld = _dense_call(
        ns_emb, adj, adj_prior, cond, labels, eps,
        Wh_post, Wq_post, Wk_post, Wv_post, Wo_post, Wmu_post, Wvar_post,
        Wh_prior, Wq_prior, Wk_prior, Wv_prior, Wo_prior, Wmu_prior,
        Wvar_prior, Wmap)

    outv, outi = _sc_topk(scores.reshape(-1))
    vals8 = lax.bitcast_convert_type(outv, jnp.float32).reshape(8, 128)
    relations = _final_call(vals8, outi.reshape(8, 128), m)
    rel_mask = jnp.zeros((MAX_K,), dtype=jnp.bool_)
    return relations, rel_mask, recons[0, 0], kld[0, 0]
